# baseline (speedup 1.0000x reference)
import functools

import jax
import jax.numpy as jnp
from jax import lax
from jax.experimental import pallas as pl
from jax.experimental.pallas import tpu as pltpu
from jax.experimental.pallas import tpu_sc as plsc

_BATCH = 16384
_EMBED = 32
_NC = 2
_NS = 16
_NW = _NC * _NS
_CHUNK = _BATCH // _NW
_L = 16
_IDXC = 128
_NIDX = _CHUNK // _IDXC
_GROUPS = _CHUNK // _L

_NUSER = 1000000
_NITEM = 100000
_PROJ_BLOCK = 131072
_USC = 4 * _PROJ_BLOCK
_UPAD = 1000448
_IPAD = 102400

_UBLK_W = _USC // 128 // _NW

_MESH = dict(core_axis_name="c", subcore_axis_name="s",
             num_cores=_NC, num_subcores=_NS)


def _proj_body(x_ref, w_ref, o_ref):
    x = x_ref[...]
    w = w_ref[...]
    o_ref[...] = jnp.sum(x * w, axis=0)


def _project_tc(table_t, w_col):
    nblk = (_UPAD - _USC + _PROJ_BLOCK - 1) // _PROJ_BLOCK
    k0 = _USC // _PROJ_BLOCK
    return pl.pallas_call(
        _proj_body,
        grid=(nblk,),
        in_specs=[
            pl.BlockSpec((_EMBED, _PROJ_BLOCK), lambda i: (0, i + k0)),
            pl.BlockSpec((_EMBED, 1), lambda i: (0, 0)),
        ],
        out_specs=pl.BlockSpec((_PROJ_BLOCK,), lambda i: (i + k0,)),
        out_shape=jax.ShapeDtypeStruct((_UPAD,), jnp.float32),
        compiler_params=pltpu.CompilerParams(
            vmem_limit_bytes=60 * 1024 * 1024),
    )(table_t, w_col)


def _project_item(table_t, w_col):
    nblk = (_IPAD + _PROJ_BLOCK - 1) // _PROJ_BLOCK
    return pl.pallas_call(
        _proj_body,
        grid=(nblk,),
        in_specs=[
            pl.BlockSpec((_EMBED, _PROJ_BLOCK), lambda i: (0, i)),
            pl.BlockSpec((_EMBED, 1), lambda i: (0, 0)),
        ],
        out_specs=pl.BlockSpec((_PROJ_BLOCK,), lambda i: (i,)),
        out_shape=jax.ShapeDtypeStruct((_IPAD,), jnp.float32),
        compiler_params=pltpu.CompilerParams(
            vmem_limit_bytes=60 * 1024 * 1024),
    )(table_t, w_col)


def _sc_proj_body(ut_hbm, w_hbm, puh_hbm,
                  xbuf, wv, out_u, sem, osem):
    wid = lax.axis_index("s") * _NC + lax.axis_index("c")
    pltpu.sync_copy(w_hbm, wv)
    w_vecs = [wv[pl.ds(q * _L, _L)] for q in range(2 * _EMBED // _L)]

    def dot_block(bb, width_chunks, w_base, out_ref, out_off):
        for l in range(width_chunks):
            acc = jnp.zeros((_L,), jnp.float32)
            for j in range(_EMBED):
                col = xbuf.at[bb, j][pl.ds(l * _L, _L)]
                wj = w_vecs[(w_base + j) // _L][(w_base + j) % _L]
                acc = acc + col * wj
            out_ref[pl.ds(out_off + l * _L, _L)] = acc

    ub0 = wid * _UBLK_W

    def u_src(k):
        return ut_hbm.at[:, pl.ds((ub0 + k) * 128, 128)]

    pltpu.async_copy(u_src(0), xbuf.at[0], sem)

    def u_loop(k, carry):
        bb = lax.rem(k, 2)
        nb = lax.rem(k + 1, 2)

        @pl.when(k + 1 < _UBLK_W)
        def _():
            pltpu.async_copy(u_src(k + 1), xbuf.at[nb], sem)

        pltpu.make_async_copy(u_src(k), xbuf.at[bb], sem).wait()
        dot_block(bb, 8, 0, out_u, k * 128)
        return carry

    lax.fori_loop(0, _UBLK_W, u_loop, 0)
    pltpu.async_copy(out_u, puh_hbm.at[pl.ds(ub0 * 128, _UBLK_W * 128)],
                     osem)
    pltpu.make_async_copy(out_u, puh_hbm.at[pl.ds(ub0 * 128,
                                                  _UBLK_W * 128)],
                          osem).wait()


def _sc_gather_body(user_idx_hbm, item_idx_hbm, puh_hbm, put_hbm, pi_hbm,
                    b_hbm, out_hbm,
                    idx_u, idx_i, idx_h, idx_t, ph_v, pt_v, pi_v, b_v,
                    out_v, sem):
    wid = lax.axis_index("s") * _NC + lax.axis_index("c")
    base = wid * _CHUNK

    pltpu.sync_copy(user_idx_hbm.at[pl.ds(wid * _NIDX, _NIDX)], idx_u)
    pltpu.sync_copy(item_idx_hbm.at[pl.ds(wid * _NIDX, _NIDX)], idx_i)
    pltpu.sync_copy(b_hbm, b_v)

    for r in range(_NIDX):
        for c in range(8):
            iu = idx_u.at[r][pl.ds(c * _L, _L)]
            in_head = iu < _USC
            idx_h.at[r][pl.ds(c * _L, _L)] = jnp.where(in_head, iu, 0)
            idx_t.at[r][pl.ds(c * _L, _L)] = jnp.where(in_head, _USC, iu)

    copies = []
    for k in range(_NIDX):
        copies.append(pltpu.async_copy(
            puh_hbm.at[idx_h.at[k]], ph_v.at[pl.ds(k * _IDXC, _IDXC)], sem))
        copies.append(pltpu.async_copy(
            put_hbm.at[idx_t.at[k]], pt_v.at[pl.ds(k * _IDXC, _IDXC)], sem))
        copies.append(pltpu.async_copy(
            pi_hbm.at[idx_i.at[k]], pi_v.at[pl.ds(k * _IDXC, _IDXC)], sem))
    for c in copies:
        c.wait()

    bias = b_v[pl.ds(0, _L)][0]

    def group(g, carry):
        r, c = g // 8, g % 8
        iu = idx_u.at[r][pl.ds(c * _L, _L)]
        pu = jnp.where(iu < _USC,
                       ph_v[pl.ds(g * _L, _L)], pt_v[pl.ds(g * _L, _L)])
        z = pu + pi_v[pl.ds(g * _L, _L)] + bias
        out_v[pl.ds(g * _L, _L)] = 1.0 / (1.0 + jnp.exp(-z))
        return carry

    for g in range(_GROUPS):
        group(g, 0)

    pltpu.sync_copy(out_v, out_hbm.at[pl.ds(base, _CHUNK)])


@jax.jit
def _recommender(user_idx, item_idx, user_table, item_table, W, b):
    ut_t = user_table.T
    it_t = item_table.T
    w_u = W[:_EMBED]
    w_flat = W.reshape(2 * _EMBED)
    b_pad = jnp.broadcast_to(b.reshape(1), (_L,))

    p_u_tail = _project_tc(ut_t, w_u)

    sc_proj = functools.partial(
        pl.kernel,
        out_type=jax.ShapeDtypeStruct((_USC,), jnp.float32),
        mesh=plsc.VectorSubcoreMesh(**_MESH),
        scratch_types=[
            pltpu.VMEM((2, _EMBED, _IDXC), jnp.float32),
            pltpu.VMEM((2 * _EMBED,), jnp.float32),
            pltpu.VMEM((_UBLK_W * 128,), jnp.float32),
            pltpu.SemaphoreType.DMA,
            pltpu.SemaphoreType.DMA,
        ],
        compiler_params=pltpu.CompilerParams(
            needs_layout_passes=False, use_tc_tiling_on_sc=True),
    )(_sc_proj_body)
    p_u_head = sc_proj(ut_t, w_flat)
    p_i = _project_item(it_t, W[_EMBED:])

    gather = functools.partial(
        pl.kernel,
        out_type=jax.ShapeDtypeStruct((_BATCH,), jnp.float32),
        mesh=plsc.VectorSubcoreMesh(**_MESH),
        scratch_types=[
            pltpu.VMEM((_NIDX, _IDXC), jnp.int32),
            pltpu.VMEM((_NIDX, _IDXC), jnp.int32),
            pltpu.VMEM((_NIDX, _IDXC), jnp.int32),
            pltpu.VMEM((_NIDX, _IDXC), jnp.int32),
            pltpu.VMEM((_CHUNK,), jnp.float32),
            pltpu.VMEM((_CHUNK,), jnp.float32),
            pltpu.VMEM((_CHUNK,), jnp.float32),
            pltpu.VMEM((_L,), jnp.float32),
            pltpu.VMEM((_CHUNK,), jnp.float32),
            pltpu.SemaphoreType.DMA,
        ],
        compiler_params=pltpu.CompilerParams(
            needs_layout_passes=False, use_tc_tiling_on_sc=False),
    )(_sc_gather_body)
    return gather(user_idx, item_idx, p_u_head, p_u_tail, p_i, b_pad)


def kernel(user_input, item_input, user_table, item_table, W, b):
    ui = user_input.astype(jnp.int32).reshape(_NW * _NIDX, _IDXC)
    ii = item_input.astype(jnp.int32).reshape(_NW * _NIDX, _IDXC)
    out = _recommender(ui, ii, user_table, item_table, W, b)
    return out.reshape(_BATCH, 1)


# speedup vs baseline: 2.0485x; 2.0485x over previous
import functools

import jax
import jax.numpy as jnp
from jax import lax
from jax.experimental import pallas as pl
from jax.experimental.pallas import tpu as pltpu
from jax.experimental.pallas import tpu_sc as plsc

_BATCH = 16384
_EMBED = 32
_NC = 2
_NS = 16
_NW = _NC * _NS
_CHUNK = _BATCH // _NW
_L = 16
_IDXC = 128
_NIDX = _CHUNK // _IDXC
_GROUPS = _CHUNK // _L

_PROJ_BLOCK = 131072


def _proj_body(x_ref, w_ref, o_ref):
    x = x_ref[...]
    w = w_ref[...]
    o_ref[...] = jnp.sum(x * w, axis=0)


def _project(table_t, w_col, n_pad):
    n = table_t.shape[1]
    grid = (n_pad + _PROJ_BLOCK - 1) // _PROJ_BLOCK
    return pl.pallas_call(
        _proj_body,
        grid=(grid,),
        in_specs=[
            pl.BlockSpec((_EMBED, _PROJ_BLOCK), lambda i: (0, i)),
            pl.BlockSpec((_EMBED, 1), lambda i: (0, 0)),
        ],
        out_specs=pl.BlockSpec((_PROJ_BLOCK,), lambda i: (i,)),
        out_shape=jax.ShapeDtypeStruct((n_pad,), jnp.float32),
        compiler_params=pltpu.CompilerParams(
            vmem_limit_bytes=60 * 1024 * 1024),
    )(table_t, w_col)


def _sc_body(user_idx_hbm, item_idx_hbm, pu_hbm, pi_hbm, b_hbm, out_hbm,
             idx_u, idx_i, pu_v, pi_v, b_v, out_v, sem):
    wid = lax.axis_index("s") * _NC + lax.axis_index("c")
    base = wid * _CHUNK

    pltpu.sync_copy(user_idx_hbm.at[pl.ds(wid * _NIDX, _NIDX)], idx_u)
    pltpu.sync_copy(item_idx_hbm.at[pl.ds(wid * _NIDX, _NIDX)], idx_i)
    pltpu.sync_copy(b_hbm, b_v)

    copies = []
    for k in range(_NIDX):
        copies.append(pltpu.async_copy(
            pu_hbm.at[idx_u.at[k]], pu_v.at[pl.ds(k * _IDXC, _IDXC)], sem))
        copies.append(pltpu.async_copy(
            pi_hbm.at[idx_i.at[k]], pi_v.at[pl.ds(k * _IDXC, _IDXC)], sem))
    for c in copies:
        c.wait()

    bias = b_v[pl.ds(0, _L)][0]

    def group(g, carry):
        z = pu_v[pl.ds(g * _L, _L)] + pi_v[pl.ds(g * _L, _L)] + bias
        out_v[pl.ds(g * _L, _L)] = 1.0 / (1.0 + jnp.exp(-z))
        return carry

    lax.fori_loop(0, _GROUPS, group, 0)

    pltpu.sync_copy(out_v, out_hbm.at[pl.ds(base, _CHUNK)])


@jax.jit
def _recommender(user_idx, item_idx, user_table, item_table, W, b):
    w_u = W[:_EMBED]
    w_i = W[_EMBED:]
    p_u = _project(user_table.T, w_u, 1000448)
    p_i = _project(item_table.T, w_i, 100352)
    b_pad = jnp.broadcast_to(b.reshape(1), (_L,))

    mesh = plsc.VectorSubcoreMesh(
        core_axis_name="c", subcore_axis_name="s",
        num_cores=_NC, num_subcores=_NS)
    run = functools.partial(
        pl.kernel,
        out_type=jax.ShapeDtypeStruct((_BATCH,), jnp.float32),
        mesh=mesh,
        scratch_types=[
            pltpu.VMEM((_NIDX, _IDXC), jnp.int32),
            pltpu.VMEM((_NIDX, _IDXC), jnp.int32),
            pltpu.VMEM((_CHUNK,), jnp.float32),
            pltpu.VMEM((_CHUNK,), jnp.float32),
            pltpu.VMEM((_L,), jnp.float32),
            pltpu.VMEM((_CHUNK,), jnp.float32),
            pltpu.SemaphoreType.DMA,
        ],
        compiler_params=pltpu.CompilerParams(
            needs_layout_passes=False, use_tc_tiling_on_sc=False),
    )(_sc_body)
    return run(user_idx, item_idx, p_u, p_i, b_pad)


def kernel(user_input, item_input, user_table, item_table, W, b):
    ui = user_input.astype(jnp.int32).reshape(_NW * _NIDX, _IDXC)
    ii = item_input.astype(jnp.int32).reshape(_NW * _NIDX, _IDXC)
    out = _recommender(ui, ii, user_table, item_table, W, b)
    return out.reshape(_BATCH, 1)
